# baseline (speedup 1.0000x reference)
import functools

import jax
import jax.numpy as jnp
from jax import lax
from jax.experimental import pallas as pl
from jax.experimental.pallas import tpu as pltpu
from jax.experimental.pallas import tpu_sc as plsc

_N = 4096
_D = 64
_BLK = 256
_NBLK = _N // _BLK

_ALPHA = 1.0

_NW = 32
_RPW = _N // _NW

_KS = (21, 33, 513)
_LEVELS = ((23, 8), (15, 8), (7, 8), (0, 7))


def _prep_body(x_ref, xb_ref, bits_ref):
    x = x_ref[...]
    xb = xb_ref[...]
    x2 = jnp.sum(x * x, axis=1)
    xb2 = jnp.sum(xb * xb, axis=1)
    g = lax.dot_general(
        xb, x, (((1,), (1,)), ((), ())), preferred_element_type=jnp.float32
    )
    d2 = xb2[:, None] + x2[None, :] - 2.0 * g
    d2c = jnp.maximum(d2, 1e-12)
    bits_ref[...] = lax.bitcast_convert_type(d2c, jnp.int32)


def _tc_prep(features):
    return pl.pallas_call(
        _prep_body,
        grid=(_NBLK,),
        in_specs=[
            pl.BlockSpec((_N, _D), lambda i: (0, 0)),
            pl.BlockSpec((_BLK, _D), lambda i: (i, 0)),
        ],
        out_specs=pl.BlockSpec((_BLK, _N), lambda i: (i, 0)),
        out_shape=jax.ShapeDtypeStruct((_N, _N), jnp.int32),
    )(features, features)


def _sc_level(row_v, hists, prefixes, ranks, shift, nbits):
    submask = (1 << nbits) - 1
    width = 1 << nbits
    nchunk_h = width // 16
    pshift = shift + nbits
    ones16 = jnp.ones((16,), jnp.int32)
    zeros16 = jnp.zeros((16,), jnp.int32)

    def zero_body(i, c):
        for h in hists:
            h[pl.ds(i * 16, 16)] = zeros16
        return c

    lax.fori_loop(0, nchunk_h, zero_body, 0)

    def pass_body(c, carry):
        x = row_v[pl.ds(c * 16, 16)]
        up = lax.shift_right_logical(x, pshift)
        sub = lax.shift_right_logical(x, shift) & submask
        for h, p in zip(hists, prefixes):
            plsc.addupdate_scatter(h, [sub], ones16, mask=(up == p))
        return carry

    lax.fori_loop(0, _N // 16, pass_body, 0, unroll=4)

    new_prefixes, new_ranks = [], []
    for h, p, r in zip(hists, prefixes, ranks):
        def scan_body(i, carry):
            tot, nbuck, below = carry
            hv = h[pl.ds(i * 16, 16)]
            cum = plsc.cumsum(hv) + tot
            m = cum < r
            nbuck = nbuck + jnp.sum(jnp.where(m, 1, 0))
            below = jnp.maximum(below, jnp.max(jnp.where(m, cum, 0)))
            tot = tot + jnp.sum(hv)
            return tot, nbuck, below

        _, b, below = lax.fori_loop(
            0, nchunk_h, scan_body,
            (jnp.int32(0), jnp.int32(0), jnp.int32(0)))
        new_prefixes.append((p << nbits) | b)
        new_ranks.append(r - below)
    return new_prefixes, new_ranks


def _sc_body(bits_hbm, thr_hbm, row_v, h0_v, h1_v, h2_v, res_v):
    wid = lax.axis_index("s") * 2 + lax.axis_index("c")
    base = wid * _RPW
    hists = (h0_v, h1_v, h2_v)

    def row_body(r, carry):
        pltpu.sync_copy(bits_hbm.at[base + r], row_v)
        prefixes = [jnp.int32(0)] * 3
        ranks = [jnp.int32(k) for k in _KS]
        for shift, nbits in _LEVELS:
            prefixes, ranks = _sc_level(
                row_v, hists, prefixes, ranks, shift, nbits)
        lanes = lax.iota(jnp.int32, 16)
        res = jnp.zeros((16,), jnp.int32)
        for j, t in enumerate(prefixes):
            res = jnp.where(lanes == j, t, res)
        res_v[pl.ds(r * 16, 16)] = res
        return carry

    lax.fori_loop(0, _RPW, row_body, 0)
    pltpu.sync_copy(res_v, thr_hbm.at[wid])


def _sc_select(bits):
    mesh = plsc.VectorSubcoreMesh(core_axis_name="c", subcore_axis_name="s")
    fn = pl.kernel(
        _sc_body,
        out_type=jax.ShapeDtypeStruct((_NW, _RPW * 16), jnp.int32),
        mesh=mesh,
        compiler_params=pltpu.CompilerParams(needs_layout_passes=False),
        scratch_types=[
            pltpu.VMEM((_N,), jnp.int32),
            pltpu.VMEM((256,), jnp.int32),
            pltpu.VMEM((256,), jnp.int32),
            pltpu.VMEM((256,), jnp.int32),
            pltpu.VMEM((_RPW * 16,), jnp.int32),
        ],
    )
    return fn(bits).reshape(_N, 16)


def _final_body(bits_ref, thr_ref, reg_ref, l32_ref, l512_ref):
    i = pl.program_id(0)
    d2c = lax.bitcast_convert_type(bits_ref[...], jnp.float32)
    thr = thr_ref[...]
    t21 = lax.bitcast_convert_type(thr[:, 0:1], jnp.float32)
    t33 = lax.bitcast_convert_type(thr[:, 1:2], jnp.float32)
    t513 = lax.bitcast_convert_type(thr[:, 2:3], jnp.float32)

    s = jnp.sqrt(d2c)
    lg = 0.5 * jnp.log(d2c)
    m21 = d2c < t21
    m33 = d2c < t33
    m513 = d2c < t513
    c21 = jnp.sum(m21.astype(jnp.float32), axis=1, keepdims=True)
    c33 = jnp.sum(m33.astype(jnp.float32), axis=1, keepdims=True)
    c513 = jnp.sum(m513.astype(jnp.float32), axis=1, keepdims=True)
    s1 = jnp.sum(jnp.where(m21, s, 0.0), axis=1, keepdims=True)
    s2 = jnp.sum(jnp.where(m33, lg, 0.0), axis=1, keepdims=True)
    s3 = jnp.sum(jnp.where(m513, lg, 0.0), axis=1, keepdims=True)
    a0 = jnp.min(s, axis=1, keepdims=True)
    log_a0 = jnp.log(a0)

    sq21 = jnp.sqrt(t21)
    lg33 = 0.5 * jnp.log(t33)
    lg513 = 0.5 * jnp.log(t513)

    s20 = s1 + (20.0 - c21) * sq21
    m = (s20 - a0) / 19.0
    lid_mom = m / (sq21 - m)
    reg_row = -jnp.abs(jnp.log(lid_mom))

    l32sum = s2 + (32.0 - c33) * lg33
    lids32 = -32.0 / (l32sum - log_a0 - 31.0 * lg33)
    l512sum = s3 + (512.0 - c513) * lg513
    lids512 = -512.0 / (l512sum - log_a0 - 511.0 * lg513)

    l32_ref[...] = lids32
    l512_ref[...] = lids512

    @pl.when(i == 0)
    def _():
        reg_ref[...] = jnp.zeros_like(reg_ref)

    reg_ref[...] += jnp.sum(reg_row, axis=(0, 1), keepdims=True)


def _tc_final(bits, thr):
    return pl.pallas_call(
        _final_body,
        grid=(_NBLK,),
        in_specs=[
            pl.BlockSpec((_BLK, _N), lambda i: (i, 0)),
            pl.BlockSpec((_BLK, 16), lambda i: (i, 0)),
        ],
        out_specs=[
            pl.BlockSpec((1, 1), lambda i: (0, 0)),
            pl.BlockSpec((_BLK, 1), lambda i: (i, 0)),
            pl.BlockSpec((_BLK, 1), lambda i: (i, 0)),
        ],
        out_shape=[
            jax.ShapeDtypeStruct((1, 1), jnp.float32),
            jax.ShapeDtypeStruct((_N, 1), jnp.float32),
            jax.ShapeDtypeStruct((_N, 1), jnp.float32),
        ],
    )(bits, thr)


def kernel(features):
    bits = _tc_prep(features)
    thr = _sc_select(bits)
    reg_sum, l32, l512 = _tc_final(bits, thr)
    reg_loss = _ALPHA * reg_sum[0, 0] / _N
    return (reg_loss, l32[:, 0], l512[:, 0])


# speedup vs baseline: 2.3306x; 2.3306x over previous
import jax
import jax.numpy as jnp
from jax import lax
from jax.experimental import pallas as pl
from jax.experimental.pallas import tpu as pltpu
from jax.experimental.pallas import tpu_sc as plsc

_N = 4096
_D = 64
_BLK = 256

_ALPHA = 1.0

_NSC = 512
_NTC = _N - _NSC
_NBLK_TC = _NTC // _BLK
_NBLK_SC = _NSC // _BLK

_NW = 32
_RPW = _NSC // _NW

_KS = (21, 33, 513)
_LEVELS = ((23, 8), (15, 8), (7, 8), (0, 7))


def _d2_bits(x, xb):
    x2 = jnp.sum(x * x, axis=1)
    xb2 = jnp.sum(xb * xb, axis=1)
    g = lax.dot_general(
        xb, x, (((1,), (1,)), ((), ())), preferred_element_type=jnp.float32
    )
    d2 = xb2[:, None] + x2[None, :] - 2.0 * g
    return jnp.maximum(d2, 1e-12)


def _estimators(d2c, t21, t33, t513):
    s = jnp.sqrt(d2c)
    lg = 0.5 * jnp.log(d2c)
    m21 = d2c < t21
    m33 = d2c < t33
    m513 = d2c < t513
    c21 = jnp.sum(m21.astype(jnp.float32), axis=1, keepdims=True)
    c33 = jnp.sum(m33.astype(jnp.float32), axis=1, keepdims=True)
    c513 = jnp.sum(m513.astype(jnp.float32), axis=1, keepdims=True)
    s1 = jnp.sum(jnp.where(m21, s, 0.0), axis=1, keepdims=True)
    s2 = jnp.sum(jnp.where(m33, lg, 0.0), axis=1, keepdims=True)
    s3 = jnp.sum(jnp.where(m513, lg, 0.0), axis=1, keepdims=True)
    a0 = jnp.min(s, axis=1, keepdims=True)
    log_a0 = jnp.log(a0)

    sq21 = jnp.sqrt(t21)
    lg33 = 0.5 * jnp.log(t33)
    lg513 = 0.5 * jnp.log(t513)

    s20 = s1 + (20.0 - c21) * sq21
    m = (s20 - a0) / 19.0
    lid_mom = m / (sq21 - m)
    reg_row = -jnp.abs(jnp.log(lid_mom))

    l32sum = s2 + (32.0 - c33) * lg33
    lids32 = -32.0 / (l32sum - log_a0 - 31.0 * lg33)
    l512sum = s3 + (512.0 - c513) * lg513
    lids512 = -512.0 / (l512sum - log_a0 - 511.0 * lg513)
    return reg_row, lids32, lids512


def _order_stats_bits(bits, ks):
    blk = bits.shape[0]
    los = [jnp.zeros((blk, 1), jnp.int32) for _ in ks]
    his = [jnp.full((blk, 1), 0x7F800000, jnp.int32) for _ in ks]

    def it(_, carry):
        los, his = carry
        nlos, nhis = [], []
        for k, lo, hi in zip(ks, los, his):
            mid = lo + (hi - lo) // 2
            cnt = jnp.sum((bits <= mid).astype(jnp.int32), axis=1, keepdims=True)
            ge = cnt >= k
            nlos.append(jnp.where(ge, lo, mid + 1))
            nhis.append(jnp.where(ge, mid, hi))
        return nlos, nhis

    los, his = lax.fori_loop(0, 31, it, (los, his))
    return los


def _main_body(x_ref, xb_ref, reg_ref, l32_ref, l512_ref):
    i = pl.program_id(0)
    d2c = _d2_bits(x_ref[...], xb_ref[...])
    bits = lax.bitcast_convert_type(d2c, jnp.int32)

    t21b, t33b, t513b = _order_stats_bits(bits, _KS)
    t21 = lax.bitcast_convert_type(t21b, jnp.float32)
    t33 = lax.bitcast_convert_type(t33b, jnp.float32)
    t513 = lax.bitcast_convert_type(t513b, jnp.float32)

    reg_row, lids32, lids512 = _estimators(d2c, t21, t33, t513)
    l32_ref[...] = lids32
    l512_ref[...] = lids512

    @pl.when(i == 0)
    def _():
        reg_ref[...] = jnp.zeros_like(reg_ref)

    reg_ref[...] += jnp.sum(reg_row, axis=(0, 1), keepdims=True)


def _tc_main(features):
    return pl.pallas_call(
        _main_body,
        grid=(_NBLK_TC,),
        in_specs=[
            pl.BlockSpec((_N, _D), lambda i: (0, 0)),
            pl.BlockSpec((_BLK, _D), lambda i: (i, 0)),
        ],
        out_specs=[
            pl.BlockSpec((1, 1), lambda i: (0, 0)),
            pl.BlockSpec((_BLK, 1), lambda i: (i, 0)),
            pl.BlockSpec((_BLK, 1), lambda i: (i, 0)),
        ],
        out_shape=[
            jax.ShapeDtypeStruct((1, 1), jnp.float32),
            jax.ShapeDtypeStruct((_NTC, 1), jnp.float32),
            jax.ShapeDtypeStruct((_NTC, 1), jnp.float32),
        ],
    )(features, features)


def _prep_body(x_ref, xb_ref, bits_ref):
    d2c = _d2_bits(x_ref[...], xb_ref[...])
    bits_ref[...] = lax.bitcast_convert_type(d2c, jnp.int32)


def _tc_prep_tail(features):
    return pl.pallas_call(
        _prep_body,
        grid=(_NBLK_SC,),
        in_specs=[
            pl.BlockSpec((_N, _D), lambda i: (0, 0)),
            pl.BlockSpec((_BLK, _D), lambda i: (i + _NBLK_TC, 0)),
        ],
        out_specs=pl.BlockSpec((_BLK, _N), lambda i: (i, 0)),
        out_shape=jax.ShapeDtypeStruct((_NSC, _N), jnp.int32),
    )(features, features)


def _sc_level(row_v, hists, prefixes, ranks, shift, nbits):
    submask = (1 << nbits) - 1
    width = 1 << nbits
    nchunk_h = width // 16
    pshift = shift + nbits
    ones16 = jnp.ones((16,), jnp.int32)
    zeros16 = jnp.zeros((16,), jnp.int32)

    def zero_body(i, c):
        for h in hists:
            h[pl.ds(i * 16, 16)] = zeros16
        return c

    lax.fori_loop(0, nchunk_h, zero_body, 0)

    def pass_body(c, carry):
        x = row_v[pl.ds(c * 16, 16)]
        up = lax.shift_right_logical(x, pshift)
        sub = lax.shift_right_logical(x, shift) & submask
        for h, p in zip(hists, prefixes):
            plsc.addupdate_scatter(h, [sub], ones16, mask=(up == p))
        return carry

    lax.fori_loop(0, _N // 16, pass_body, 0, unroll=4)

    new_prefixes, new_ranks = [], []
    for h, p, r in zip(hists, prefixes, ranks):
        def scan_body(i, carry):
            tot, nbuck, below = carry
            hv = h[pl.ds(i * 16, 16)]
            cum = plsc.cumsum(hv) + tot
            m = cum < r
            nbuck = nbuck + jnp.sum(jnp.where(m, 1, 0))
            below = jnp.maximum(below, jnp.max(jnp.where(m, cum, 0)))
            tot = tot + jnp.sum(hv)
            return tot, nbuck, below

        _, b, below = lax.fori_loop(
            0, nchunk_h, scan_body,
            (jnp.int32(0), jnp.int32(0), jnp.int32(0)))
        new_prefixes.append((p << nbits) | b)
        new_ranks.append(r - below)
    return new_prefixes, new_ranks


def _sc_body(bits_hbm, thr_hbm, row_v, h0_v, h1_v, h2_v, res_v):
    wid = lax.axis_index("s") * 2 + lax.axis_index("c")
    base = wid * _RPW
    hists = (h0_v, h1_v, h2_v)

    def row_body(r, carry):
        pltpu.sync_copy(bits_hbm.at[base + r], row_v)
        prefixes = [jnp.int32(0)] * 3
        ranks = [jnp.int32(k) for k in _KS]
        for shift, nbits in _LEVELS:
            prefixes, ranks = _sc_level(
                row_v, hists, prefixes, ranks, shift, nbits)
        lanes = lax.iota(jnp.int32, 16)
        res = jnp.zeros((16,), jnp.int32)
        for j, t in enumerate(prefixes):
            res = jnp.where(lanes == j, t, res)
        res_v[pl.ds(r * 16, 16)] = res
        return carry

    lax.fori_loop(0, _RPW, row_body, 0)
    pltpu.sync_copy(res_v, thr_hbm.at[wid])


def _sc_select(bits):
    mesh = plsc.VectorSubcoreMesh(core_axis_name="c", subcore_axis_name="s")
    fn = pl.kernel(
        _sc_body,
        out_type=jax.ShapeDtypeStruct((_NW, _RPW * 16), jnp.int32),
        mesh=mesh,
        compiler_params=pltpu.CompilerParams(needs_layout_passes=False),
        scratch_types=[
            pltpu.VMEM((_N,), jnp.int32),
            pltpu.VMEM((256,), jnp.int32),
            pltpu.VMEM((256,), jnp.int32),
            pltpu.VMEM((256,), jnp.int32),
            pltpu.VMEM((_RPW * 16,), jnp.int32),
        ],
    )
    return fn(bits).reshape(_NSC, 16)


def _tail_body(bits_ref, thr_ref, reg_ref, l32_ref, l512_ref):
    i = pl.program_id(0)
    d2c = lax.bitcast_convert_type(bits_ref[...], jnp.float32)
    thr = thr_ref[...]
    t21 = lax.bitcast_convert_type(thr[:, 0:1], jnp.float32)
    t33 = lax.bitcast_convert_type(thr[:, 1:2], jnp.float32)
    t513 = lax.bitcast_convert_type(thr[:, 2:3], jnp.float32)

    reg_row, lids32, lids512 = _estimators(d2c, t21, t33, t513)
    l32_ref[...] = lids32
    l512_ref[...] = lids512

    @pl.when(i == 0)
    def _():
        reg_ref[...] = jnp.zeros_like(reg_ref)

    reg_ref[...] += jnp.sum(reg_row, axis=(0, 1), keepdims=True)


def _tc_tail(bits, thr):
    return pl.pallas_call(
        _tail_body,
        grid=(_NBLK_SC,),
        in_specs=[
            pl.BlockSpec((_BLK, _N), lambda i: (i, 0)),
            pl.BlockSpec((_BLK, 16), lambda i: (i, 0)),
        ],
        out_specs=[
            pl.BlockSpec((1, 1), lambda i: (0, 0)),
            pl.BlockSpec((_BLK, 1), lambda i: (i, 0)),
            pl.BlockSpec((_BLK, 1), lambda i: (i, 0)),
        ],
        out_shape=[
            jax.ShapeDtypeStruct((1, 1), jnp.float32),
            jax.ShapeDtypeStruct((_NSC, 1), jnp.float32),
            jax.ShapeDtypeStruct((_NSC, 1), jnp.float32),
        ],
    )(bits, thr)


def kernel(features):
    bits_tail = _tc_prep_tail(features)
    thr_tail = _sc_select(bits_tail)
    reg_main, l32_m, l512_m = _tc_main(features)
    reg_tail, l32_t, l512_t = _tc_tail(bits_tail, thr_tail)
    reg_loss = _ALPHA * (reg_main[0, 0] + reg_tail[0, 0]) / _N
    l32 = jnp.concatenate([l32_m[:, 0], l32_t[:, 0]])
    l512 = jnp.concatenate([l512_m[:, 0], l512_t[:, 0]])
    return (reg_loss, l32, l512)


# speedup vs baseline: 2.8485x; 1.2222x over previous
import jax
import jax.numpy as jnp
from jax import lax
from jax.experimental import pallas as pl
from jax.experimental.pallas import tpu as pltpu
from jax.experimental.pallas import tpu_sc as plsc

_N = 4096
_D = 64
_BLK = 256

_ALPHA = 1.0

_NSC = 1280
_NTC = _N - _NSC
_NBLK_TC = _NTC // _BLK
_NBLK_SC = _NSC // _BLK

_NW = 32
_RPW = _NSC // _NW

_KS = (21, 33, 513)
_LEVELS = ((23, 8), (15, 8), (7, 8), (0, 7))


def _d2_bits(x, xb):
    x2 = jnp.sum(x * x, axis=1)
    xb2 = jnp.sum(xb * xb, axis=1)
    g = lax.dot_general(
        xb, x, (((1,), (1,)), ((), ())), preferred_element_type=jnp.float32
    )
    d2 = xb2[:, None] + x2[None, :] - 2.0 * g
    return jnp.maximum(d2, 1e-12)


def _estimators(d2c, t21, t33, t513):
    s = jnp.sqrt(d2c)
    lg = 0.5 * jnp.log(d2c)
    m21 = d2c < t21
    m33 = d2c < t33
    m513 = d2c < t513
    c21 = jnp.sum(m21.astype(jnp.float32), axis=1, keepdims=True)
    c33 = jnp.sum(m33.astype(jnp.float32), axis=1, keepdims=True)
    c513 = jnp.sum(m513.astype(jnp.float32), axis=1, keepdims=True)
    s1 = jnp.sum(jnp.where(m21, s, 0.0), axis=1, keepdims=True)
    s2 = jnp.sum(jnp.where(m33, lg, 0.0), axis=1, keepdims=True)
    s3 = jnp.sum(jnp.where(m513, lg, 0.0), axis=1, keepdims=True)
    a0 = jnp.min(s, axis=1, keepdims=True)
    log_a0 = jnp.log(a0)

    sq21 = jnp.sqrt(t21)
    lg33 = 0.5 * jnp.log(t33)
    lg513 = 0.5 * jnp.log(t513)

    s20 = s1 + (20.0 - c21) * sq21
    m = (s20 - a0) / 19.0
    lid_mom = m / (sq21 - m)
    reg_row = -jnp.abs(jnp.log(lid_mom))

    l32sum = s2 + (32.0 - c33) * lg33
    lids32 = -32.0 / (l32sum - log_a0 - 31.0 * lg33)
    l512sum = s3 + (512.0 - c513) * lg513
    lids512 = -512.0 / (l512sum - log_a0 - 511.0 * lg513)
    return reg_row, lids32, lids512


def _order_stats_bits(bits, ks):
    blk = bits.shape[0]
    los = [jnp.zeros((blk, 1), jnp.int32) for _ in ks]
    his = [jnp.full((blk, 1), 0x7F800000, jnp.int32) for _ in ks]

    def it(_, carry):
        los, his = carry
        nlos, nhis = [], []
        for k, lo, hi in zip(ks, los, his):
            mid = lo + (hi - lo) // 2
            cnt = jnp.sum((bits <= mid).astype(jnp.int32), axis=1, keepdims=True)
            ge = cnt >= k
            nlos.append(jnp.where(ge, lo, mid + 1))
            nhis.append(jnp.where(ge, mid, hi))
        return nlos, nhis

    los, his = lax.fori_loop(0, 31, it, (los, his))
    return los


def _main_body(x_ref, xb_ref, reg_ref, l32_ref, l512_ref):
    i = pl.program_id(0)
    d2c = _d2_bits(x_ref[...], xb_ref[...])
    bits = lax.bitcast_convert_type(d2c, jnp.int32)

    t21b, t33b, t513b = _order_stats_bits(bits, _KS)
    t21 = lax.bitcast_convert_type(t21b, jnp.float32)
    t33 = lax.bitcast_convert_type(t33b, jnp.float32)
    t513 = lax.bitcast_convert_type(t513b, jnp.float32)

    reg_row, lids32, lids512 = _estimators(d2c, t21, t33, t513)
    l32_ref[...] = lids32
    l512_ref[...] = lids512

    @pl.when(i == 0)
    def _():
        reg_ref[...] = jnp.zeros_like(reg_ref)

    reg_ref[...] += jnp.sum(reg_row, axis=(0, 1), keepdims=True)


def _tc_main(features):
    return pl.pallas_call(
        _main_body,
        grid=(_NBLK_TC,),
        in_specs=[
            pl.BlockSpec((_N, _D), lambda i: (0, 0)),
            pl.BlockSpec((_BLK, _D), lambda i: (i, 0)),
        ],
        out_specs=[
            pl.BlockSpec((1, 1), lambda i: (0, 0)),
            pl.BlockSpec((_BLK, 1), lambda i: (i, 0)),
            pl.BlockSpec((_BLK, 1), lambda i: (i, 0)),
        ],
        out_shape=[
            jax.ShapeDtypeStruct((1, 1), jnp.float32),
            jax.ShapeDtypeStruct((_NTC, 1), jnp.float32),
            jax.ShapeDtypeStruct((_NTC, 1), jnp.float32),
        ],
    )(features, features)


def _prep_body(x_ref, xb_ref, bits_ref):
    d2c = _d2_bits(x_ref[...], xb_ref[...])
    bits_ref[...] = lax.bitcast_convert_type(d2c, jnp.int32)


def _tc_prep_tail(features):
    return pl.pallas_call(
        _prep_body,
        grid=(_NBLK_SC,),
        in_specs=[
            pl.BlockSpec((_N, _D), lambda i: (0, 0)),
            pl.BlockSpec((_BLK, _D), lambda i: (i + _NBLK_TC, 0)),
        ],
        out_specs=pl.BlockSpec((_BLK, _N), lambda i: (i, 0)),
        out_shape=jax.ShapeDtypeStruct((_NSC, _N), jnp.int32),
    )(features, features)


def _sc_level(row_v, hists, prefixes, ranks, shift, nbits):
    submask = (1 << nbits) - 1
    width = 1 << nbits
    nchunk_h = width // 16
    pshift = shift + nbits
    ones16 = jnp.ones((16,), jnp.int32)
    zeros16 = jnp.zeros((16,), jnp.int32)

    def zero_body(i, c):
        for h in hists:
            h[pl.ds(i * 16, 16)] = zeros16
        return c

    lax.fori_loop(0, nchunk_h, zero_body, 0)

    shared = len(hists) == 1

    def pass_body(c, carry):
        x = row_v[pl.ds(c * 16, 16)]
        sub = lax.shift_right_logical(x, shift) & submask
        if shared:
            plsc.addupdate_scatter(hists[0], [sub], ones16)
        else:
            up = lax.shift_right_logical(x, pshift)
            for h, p in zip(hists, prefixes):
                plsc.addupdate_scatter(h, [sub], ones16, mask=(up == p))
        return carry

    lax.fori_loop(0, _N // 16, pass_body, 0, unroll=8)

    scan_hists = hists * 3 if shared else hists
    new_prefixes, new_ranks = [], []
    for h, p, r in zip(scan_hists, prefixes, ranks):
        def scan_body(i, carry):
            tot, nbuck, below = carry
            hv = h[pl.ds(i * 16, 16)]
            cum = plsc.cumsum(hv) + tot
            m = cum < r
            nbuck = nbuck + jnp.sum(jnp.where(m, 1, 0))
            below = jnp.maximum(below, jnp.max(jnp.where(m, cum, 0)))
            tot = tot + jnp.sum(hv)
            return tot, nbuck, below

        _, b, below = lax.fori_loop(
            0, nchunk_h, scan_body,
            (jnp.int32(0), jnp.int32(0), jnp.int32(0)))
        new_prefixes.append((p << nbits) | b)
        new_ranks.append(r - below)
    return new_prefixes, new_ranks


def _sc_body(bits_hbm, thr_hbm, row_v, h0_v, h1_v, h2_v, res_v):
    wid = lax.axis_index("s") * 2 + lax.axis_index("c")
    base = wid * _RPW
    hists = (h0_v, h1_v, h2_v)

    def row_body(r, carry):
        pltpu.sync_copy(bits_hbm.at[base + r], row_v)
        prefixes = [jnp.int32(0)] * 3
        ranks = [jnp.int32(k) for k in _KS]
        for lvl, (shift, nbits) in enumerate(_LEVELS):
            lvl_hists = hists[:1] if lvl == 0 else hists
            prefixes, ranks = _sc_level(
                row_v, lvl_hists, prefixes, ranks, shift, nbits)
        lanes = lax.iota(jnp.int32, 16)
        res = jnp.zeros((16,), jnp.int32)
        for j, t in enumerate(prefixes):
            res = jnp.where(lanes == j, t, res)
        res_v[pl.ds(r * 16, 16)] = res
        return carry

    lax.fori_loop(0, _RPW, row_body, 0)
    pltpu.sync_copy(res_v, thr_hbm.at[wid])


def _sc_select(bits):
    mesh = plsc.VectorSubcoreMesh(core_axis_name="c", subcore_axis_name="s")
    fn = pl.kernel(
        _sc_body,
        out_type=jax.ShapeDtypeStruct((_NW, _RPW * 16), jnp.int32),
        mesh=mesh,
        compiler_params=pltpu.CompilerParams(needs_layout_passes=False),
        scratch_types=[
            pltpu.VMEM((_N,), jnp.int32),
            pltpu.VMEM((256,), jnp.int32),
            pltpu.VMEM((256,), jnp.int32),
            pltpu.VMEM((256,), jnp.int32),
            pltpu.VMEM((_RPW * 16,), jnp.int32),
        ],
    )
    return fn(bits).reshape(_NSC, 16)


def _tail_body(bits_ref, thr_ref, reg_ref, l32_ref, l512_ref):
    i = pl.program_id(0)
    d2c = lax.bitcast_convert_type(bits_ref[...], jnp.float32)
    thr = thr_ref[...]
    t21 = lax.bitcast_convert_type(thr[:, 0:1], jnp.float32)
    t33 = lax.bitcast_convert_type(thr[:, 1:2], jnp.float32)
    t513 = lax.bitcast_convert_type(thr[:, 2:3], jnp.float32)

    reg_row, lids32, lids512 = _estimators(d2c, t21, t33, t513)
    l32_ref[...] = lids32
    l512_ref[...] = lids512

    @pl.when(i == 0)
    def _():
        reg_ref[...] = jnp.zeros_like(reg_ref)

    reg_ref[...] += jnp.sum(reg_row, axis=(0, 1), keepdims=True)


def _tc_tail(bits, thr):
    return pl.pallas_call(
        _tail_body,
        grid=(_NBLK_SC,),
        in_specs=[
            pl.BlockSpec((_BLK, _N), lambda i: (i, 0)),
            pl.BlockSpec((_BLK, 16), lambda i: (i, 0)),
        ],
        out_specs=[
            pl.BlockSpec((1, 1), lambda i: (0, 0)),
            pl.BlockSpec((_BLK, 1), lambda i: (i, 0)),
            pl.BlockSpec((_BLK, 1), lambda i: (i, 0)),
        ],
        out_shape=[
            jax.ShapeDtypeStruct((1, 1), jnp.float32),
            jax.ShapeDtypeStruct((_NSC, 1), jnp.float32),
            jax.ShapeDtypeStruct((_NSC, 1), jnp.float32),
        ],
    )(bits, thr)


def kernel(features):
    bits_tail = _tc_prep_tail(features)
    thr_tail = _sc_select(bits_tail)
    reg_main, l32_m, l512_m = _tc_main(features)
    reg_tail, l32_t, l512_t = _tc_tail(bits_tail, thr_tail)
    reg_loss = _ALPHA * (reg_main[0, 0] + reg_tail[0, 0]) / _N
    l32 = jnp.concatenate([l32_m[:, 0], l32_t[:, 0]])
    l512 = jnp.concatenate([l512_m[:, 0], l512_t[:, 0]])
    return (reg_loss, l32, l512)


# speedup vs baseline: 3.0739x; 1.0791x over previous
import jax
import jax.numpy as jnp
from jax import lax
from jax.experimental import pallas as pl
from jax.experimental.pallas import tpu as pltpu
from jax.experimental.pallas import tpu_sc as plsc

_N = 4096
_D = 64
_BLK = 256

_ALPHA = 1.0

_NSC = 1536
_NTC = _N - _NSC
_NBLK_TC = _NTC // _BLK
_NBLK_SC = _NSC // _BLK

_NW = 32
_RPW = _NSC // _NW

_KS = (21, 33, 513)
_LEVELS = ((23, 8), (15, 8), (7, 8), (0, 7))


def _d2_bits(x, xb):
    x2 = jnp.sum(x * x, axis=1)
    xb2 = jnp.sum(xb * xb, axis=1)
    g = lax.dot_general(
        xb, x, (((1,), (1,)), ((), ())), preferred_element_type=jnp.float32
    )
    d2 = xb2[:, None] + x2[None, :] - 2.0 * g
    return jnp.maximum(d2, 1e-12)


def _estimators(d2c, t21, t33, t513):
    s = jnp.sqrt(d2c)
    lg = 0.5 * jnp.log(d2c)
    m21 = d2c < t21
    m33 = d2c < t33
    m513 = d2c < t513
    c21 = jnp.sum(m21.astype(jnp.float32), axis=1, keepdims=True)
    c33 = jnp.sum(m33.astype(jnp.float32), axis=1, keepdims=True)
    c513 = jnp.sum(m513.astype(jnp.float32), axis=1, keepdims=True)
    s1 = jnp.sum(jnp.where(m21, s, 0.0), axis=1, keepdims=True)
    s2 = jnp.sum(jnp.where(m33, lg, 0.0), axis=1, keepdims=True)
    s3 = jnp.sum(jnp.where(m513, lg, 0.0), axis=1, keepdims=True)
    a0 = jnp.min(s, axis=1, keepdims=True)
    log_a0 = jnp.log(a0)

    sq21 = jnp.sqrt(t21)
    lg33 = 0.5 * jnp.log(t33)
    lg513 = 0.5 * jnp.log(t513)

    s20 = s1 + (20.0 - c21) * sq21
    m = (s20 - a0) / 19.0
    lid_mom = m / (sq21 - m)
    reg_row = -jnp.abs(jnp.log(lid_mom))

    l32sum = s2 + (32.0 - c33) * lg33
    lids32 = -32.0 / (l32sum - log_a0 - 31.0 * lg33)
    l512sum = s3 + (512.0 - c513) * lg513
    lids512 = -512.0 / (l512sum - log_a0 - 511.0 * lg513)
    return reg_row, lids32, lids512


def _order_stats_bits(bits, ks):
    blk = bits.shape[0]
    los = [jnp.zeros((blk, 1), jnp.int32) for _ in ks]
    his = [jnp.full((blk, 1), 0x7F800000, jnp.int32) for _ in ks]

    def it(_, carry):
        los, his = carry
        nlos, nhis = [], []
        for k, lo, hi in zip(ks, los, his):
            mid = lo + (hi - lo) // 2
            cnt = jnp.sum((bits <= mid).astype(jnp.int32), axis=1, keepdims=True)
            ge = cnt >= k
            nlos.append(jnp.where(ge, lo, mid + 1))
            nhis.append(jnp.where(ge, mid, hi))
        return nlos, nhis

    los, his = lax.fori_loop(0, 31, it, (los, his))
    return los


def _main_body(x_ref, xb_ref, reg_ref, l32_ref, l512_ref):
    i = pl.program_id(0)
    d2c = _d2_bits(x_ref[...], xb_ref[...])
    bits = lax.bitcast_convert_type(d2c, jnp.int32)

    t21b, t33b, t513b = _order_stats_bits(bits, _KS)
    t21 = lax.bitcast_convert_type(t21b, jnp.float32)
    t33 = lax.bitcast_convert_type(t33b, jnp.float32)
    t513 = lax.bitcast_convert_type(t513b, jnp.float32)

    reg_row, lids32, lids512 = _estimators(d2c, t21, t33, t513)
    l32_ref[...] = lids32
    l512_ref[...] = lids512

    @pl.when(i == 0)
    def _():
        reg_ref[...] = jnp.zeros_like(reg_ref)

    reg_ref[...] += jnp.sum(reg_row, axis=(0, 1), keepdims=True)


def _tc_main(features):
    return pl.pallas_call(
        _main_body,
        grid=(_NBLK_TC,),
        in_specs=[
            pl.BlockSpec((_N, _D), lambda i: (0, 0)),
            pl.BlockSpec((_BLK, _D), lambda i: (i, 0)),
        ],
        out_specs=[
            pl.BlockSpec((1, 1), lambda i: (0, 0)),
            pl.BlockSpec((_BLK, 1), lambda i: (i, 0)),
            pl.BlockSpec((_BLK, 1), lambda i: (i, 0)),
        ],
        out_shape=[
            jax.ShapeDtypeStruct((1, 1), jnp.float32),
            jax.ShapeDtypeStruct((_NTC, 1), jnp.float32),
            jax.ShapeDtypeStruct((_NTC, 1), jnp.float32),
        ],
    )(features, features)


def _prep_body(x_ref, xb_ref, bits_ref):
    d2c = _d2_bits(x_ref[...], xb_ref[...])
    bits_ref[...] = lax.bitcast_convert_type(d2c, jnp.int32)


def _tc_prep_tail(features):
    return pl.pallas_call(
        _prep_body,
        grid=(_NBLK_SC,),
        in_specs=[
            pl.BlockSpec((_N, _D), lambda i: (0, 0)),
            pl.BlockSpec((_BLK, _D), lambda i: (i + _NBLK_TC, 0)),
        ],
        out_specs=pl.BlockSpec((_BLK, _N), lambda i: (i, 0)),
        out_shape=jax.ShapeDtypeStruct((_NSC, _N), jnp.int32),
    )(features, features)


def _sc_level(row_v, hists, prefixes, ranks, shift, nbits):
    submask = (1 << nbits) - 1
    width = 1 << nbits
    nchunk_h = width // 16
    pshift = shift + nbits
    ones16 = jnp.ones((16,), jnp.int32)
    zeros16 = jnp.zeros((16,), jnp.int32)

    def zero_body(i, c):
        for h in hists:
            h[pl.ds(i * 16, 16)] = zeros16
        return c

    lax.fori_loop(0, nchunk_h, zero_body, 0)

    shared = len(hists) == 1

    def pass_body(c, carry):
        x = row_v[pl.ds(c * 16, 16)]
        sub = lax.shift_right_logical(x, shift) & submask
        if shared:
            plsc.addupdate_scatter(hists[0], [sub], ones16)
        else:
            up = lax.shift_right_logical(x, pshift)
            for h, p in zip(hists, prefixes):
                plsc.addupdate_scatter(h, [sub], ones16, mask=(up == p))
        return carry

    lax.fori_loop(0, _N // 16, pass_body, 0, unroll=8)

    scan_hists = hists * 3 if shared else hists
    new_prefixes, new_ranks = [], []
    for h, p, r in zip(scan_hists, prefixes, ranks):
        def scan_body(i, carry):
            tot, nbuck, below = carry
            hv = h[pl.ds(i * 16, 16)]
            cum = plsc.cumsum(hv) + tot
            m = cum < r
            nbuck = nbuck + jnp.sum(jnp.where(m, 1, 0))
            below = jnp.maximum(below, jnp.max(jnp.where(m, cum, 0)))
            tot = tot + jnp.sum(hv)
            return tot, nbuck, below

        _, b, below = lax.fori_loop(
            0, nchunk_h, scan_body,
            (jnp.int32(0), jnp.int32(0), jnp.int32(0)))
        new_prefixes.append((p << nbits) | b)
        new_ranks.append(r - below)
    return new_prefixes, new_ranks


def _sc_body(bits_hbm, thr_hbm, row_v, h0_v, h1_v, h2_v, res_v):
    wid = lax.axis_index("s") * 2 + lax.axis_index("c")
    base = wid * _RPW
    hists = (h0_v, h1_v, h2_v)

    def row_body(r, carry):
        pltpu.sync_copy(bits_hbm.at[base + r], row_v)
        prefixes = [jnp.int32(0)] * 3
        ranks = [jnp.int32(k) for k in _KS]
        for lvl, (shift, nbits) in enumerate(_LEVELS):
            lvl_hists = hists[:1] if lvl == 0 else hists
            prefixes, ranks = _sc_level(
                row_v, lvl_hists, prefixes, ranks, shift, nbits)
        lanes = lax.iota(jnp.int32, 16)
        res = jnp.zeros((16,), jnp.int32)
        for j, t in enumerate(prefixes):
            res = jnp.where(lanes == j, t, res)
        res_v[pl.ds(r * 16, 16)] = res
        return carry

    lax.fori_loop(0, _RPW, row_body, 0)
    pltpu.sync_copy(res_v, thr_hbm.at[wid])


def _sc_select(bits):
    mesh = plsc.VectorSubcoreMesh(core_axis_name="c", subcore_axis_name="s")
    fn = pl.kernel(
        _sc_body,
        out_type=jax.ShapeDtypeStruct((_NW, _RPW * 16), jnp.int32),
        mesh=mesh,
        compiler_params=pltpu.CompilerParams(needs_layout_passes=False),
        scratch_types=[
            pltpu.VMEM((_N,), jnp.int32),
            pltpu.VMEM((256,), jnp.int32),
            pltpu.VMEM((256,), jnp.int32),
            pltpu.VMEM((256,), jnp.int32),
            pltpu.VMEM((_RPW * 16,), jnp.int32),
        ],
    )
    return fn(bits).reshape(_NSC, 16)


def _tail_body(bits_ref, thr_ref, reg_ref, l32_ref, l512_ref):
    i = pl.program_id(0)
    d2c = lax.bitcast_convert_type(bits_ref[...], jnp.float32)
    thr = thr_ref[...]
    t21 = lax.bitcast_convert_type(thr[:, 0:1], jnp.float32)
    t33 = lax.bitcast_convert_type(thr[:, 1:2], jnp.float32)
    t513 = lax.bitcast_convert_type(thr[:, 2:3], jnp.float32)

    reg_row, lids32, lids512 = _estimators(d2c, t21, t33, t513)
    l32_ref[...] = lids32
    l512_ref[...] = lids512

    @pl.when(i == 0)
    def _():
        reg_ref[...] = jnp.zeros_like(reg_ref)

    reg_ref[...] += jnp.sum(reg_row, axis=(0, 1), keepdims=True)


def _tc_tail(bits, thr):
    return pl.pallas_call(
        _tail_body,
        grid=(_NBLK_SC,),
        in_specs=[
            pl.BlockSpec((_BLK, _N), lambda i: (i, 0)),
            pl.BlockSpec((_BLK, 16), lambda i: (i, 0)),
        ],
        out_specs=[
            pl.BlockSpec((1, 1), lambda i: (0, 0)),
            pl.BlockSpec((_BLK, 1), lambda i: (i, 0)),
            pl.BlockSpec((_BLK, 1), lambda i: (i, 0)),
        ],
        out_shape=[
            jax.ShapeDtypeStruct((1, 1), jnp.float32),
            jax.ShapeDtypeStruct((_NSC, 1), jnp.float32),
            jax.ShapeDtypeStruct((_NSC, 1), jnp.float32),
        ],
    )(bits, thr)


def kernel(features):
    bits_tail = _tc_prep_tail(features)
    thr_tail = _sc_select(bits_tail)
    reg_main, l32_m, l512_m = _tc_main(features)
    reg_tail, l32_t, l512_t = _tc_tail(bits_tail, thr_tail)
    reg_loss = _ALPHA * (reg_main[0, 0] + reg_tail[0, 0]) / _N
    l32 = jnp.concatenate([l32_m[:, 0], l32_t[:, 0]])
    l512 = jnp.concatenate([l512_m[:, 0], l512_t[:, 0]])
    return (reg_loss, l32, l512)
